# Initial kernel scaffold; baseline (speedup 1.0000x reference)
import jax
import jax.numpy as jnp
from jax.experimental import pallas as pl


def kernel(x, edge_index_l0, edge_index_l1, pos_edge_index, neg_edge_index, W_in, b_in, lora_A, lora_B, pn_g, pn_b, Ws0, Wn0, bb0, ln0_g, ln0_b, Ws1, Wn1, bb1, ln1_g, ln1_b):
    raise NotImplementedError("write your pallas kernel here")


# speedup vs baseline: 4.7831x; 4.7831x over previous
import functools

import jax
import jax.numpy as jnp
from jax import lax
from jax.experimental import pallas as pl
from jax.experimental.pallas import tpu as pltpu
from jax.experimental.pallas import tpu_sc as plsc

N = 10000
D = 128
H = 128
E = 320000
EP = 10000

NC = 2
NS = 16
NW = NC * NS
K = 80

EDGES_PER_CORE = E // NC
EDGES_PER_TILE = E // NW
CHUNKS_PER_TILE = EDGES_PER_TILE // K

ZROWS = 16
NROWCHUNKS = N // ZROWS


def _sc_aggregate_body(h_hbm, src_hbm, dst_hbm, agg_out, deg_out,
                       src_v, dst_v, ones_v, rows_v, zrow_v, zdeg_v,
                       agg_sh, deg_sh, sem):
    c = lax.axis_index("c")
    s = lax.axis_index("s")

    zero16 = jnp.zeros((16,), jnp.float32)
    one16 = jnp.ones((16,), jnp.float32)
    for j in range(K // 16):
        ones_v[pl.ds(j * 16, 16)] = one16
    for r in range(ZROWS):
        for q in range(H // 16):
            zrow_v[r, pl.ds(q * 16, 16)] = zero16
    for j in range(2000 // 16):
        zdeg_v[pl.ds(j * 16, 16)] = zero16

    def zbody(k, carry):
        cid = s + k * NS

        @pl.when(cid < NROWCHUNKS)
        def _():
            pltpu.sync_copy(zrow_v, agg_sh.at[pl.ds(cid * ZROWS, ZROWS)])

        return carry

    lax.fori_loop(0, (NROWCHUNKS + NS - 1) // NS, zbody, 0)

    @pl.when(s < 5)
    def _zero_deg():
        pltpu.sync_copy(zdeg_v, deg_sh.at[pl.ds(s * 2000, 2000)])

    plsc.subcore_barrier()

    estart = c * EDGES_PER_CORE + s * EDGES_PER_TILE

    def chunk(i, carry):
        base = estart + i * K
        pltpu.sync_copy(src_hbm.at[pl.ds(base, K)], src_v)
        pltpu.sync_copy(dst_hbm.at[pl.ds(base, K)], dst_v)
        pltpu.async_copy(h_hbm.at[src_v], rows_v, sem).wait()
        pltpu.sync_copy(rows_v, agg_sh.at[dst_v], add=True)
        pltpu.sync_copy(ones_v, deg_sh.at[dst_v], add=True)
        return carry

    lax.fori_loop(0, CHUNKS_PER_TILE, chunk, 0)

    plsc.subcore_barrier()

    def wbody(k, carry):
        cid = s + k * NS

        @pl.when(cid < NROWCHUNKS)
        def _():
            pltpu.sync_copy(agg_sh.at[pl.ds(cid * ZROWS, ZROWS)], zrow_v)
            pltpu.sync_copy(zrow_v, agg_out.at[c, pl.ds(cid * ZROWS, ZROWS)])

        return carry

    lax.fori_loop(0, (NROWCHUNKS + NS - 1) // NS, wbody, 0)

    @pl.when(s < 5)
    def _write_deg():
        pltpu.sync_copy(deg_sh.at[pl.ds(s * 2000, 2000)], zdeg_v)
        pltpu.sync_copy(zdeg_v, deg_out.at[pl.ds(c * N + s * 2000, 2000)])


def _sc_aggregate(h, src_arr, dst_arr):
    mesh = plsc.VectorSubcoreMesh(core_axis_name="c", subcore_axis_name="s")
    return pl.kernel(
        _sc_aggregate_body,
        out_type=[jax.ShapeDtypeStruct((NC, N, H), jnp.float32),
                  jax.ShapeDtypeStruct((NC * N,), jnp.float32)],
        mesh=mesh,
        scratch_types=[
            pltpu.VMEM((K,), jnp.int32),
            pltpu.VMEM((K,), jnp.int32),
            pltpu.VMEM((K,), jnp.float32),
            pltpu.VMEM((K, H), jnp.float32),
            pltpu.VMEM((ZROWS, H), jnp.float32),
            pltpu.VMEM((2000,), jnp.float32),
            pltpu.VMEM_SHARED((N, H), jnp.float32),
            pltpu.VMEM_SHARED((N,), jnp.float32),
            pltpu.SemaphoreType.DMA,
        ],
    )(h, src_arr, dst_arr)


NCHUNKS_SCORE = 2 * EP // K
CHUNKS_PER_ARRAY = EP // K


def _sc_scores_body(h_hbm, comb_hbm, prows_out,
                    aidx_v, bidx_v, rows_a, rows_b, sem_a, sem_b):
    c = lax.axis_index("c")
    s = lax.axis_index("s")
    w = s * NC + c

    def do_chunk(cid):
        g = cid // CHUNKS_PER_ARRAY
        off = (cid % CHUNKS_PER_ARRAY) * K
        pltpu.sync_copy(comb_hbm.at[pl.ds(2 * g * EP + off, K)], aidx_v)
        pltpu.sync_copy(comb_hbm.at[pl.ds((2 * g + 1) * EP + off, K)], bidx_v)
        cp_a = pltpu.async_copy(h_hbm.at[aidx_v], rows_a, sem_a)
        cp_b = pltpu.async_copy(h_hbm.at[bidx_v], rows_b, sem_b)
        cp_a.wait()
        cp_b.wait()

        def ebody(e, carry):
            for q in range(H // 16):
                sl = pl.ds(q * 16, 16)
                rows_a[e, sl] = rows_a[e, sl] * rows_b[e, sl]
            return carry

        lax.fori_loop(0, K, ebody, 0)
        pltpu.sync_copy(rows_a, prows_out.at[pl.ds(g * EP + off, K)])

    def loop(k, carry):
        cid = w + k * NW

        @pl.when(cid < NCHUNKS_SCORE)
        def _():
            do_chunk(cid)

        return carry

    nmax = (NCHUNKS_SCORE + NW - 1) // NW
    lax.fori_loop(0, nmax, loop, 0)


def _sc_scores(h, comb):
    mesh = plsc.VectorSubcoreMesh(core_axis_name="c", subcore_axis_name="s")
    return pl.kernel(
        _sc_scores_body,
        out_type=jax.ShapeDtypeStruct((2 * EP, H), jnp.float32),
        mesh=mesh,
        scratch_types=[
            pltpu.VMEM((K,), jnp.int32),
            pltpu.VMEM((K,), jnp.int32),
            pltpu.VMEM((K, H), jnp.float32),
            pltpu.VMEM((K, H), jnp.float32),
            pltpu.SemaphoreType.DMA,
            pltpu.SemaphoreType.DMA,
        ],
    )(h, comb)


def _dots_body(p_ref, out_ref):
    out_ref[...] = jnp.sum(p_ref[...], axis=-1, keepdims=True)


def _dots_tc(prows):
    return pl.pallas_call(
        _dots_body,
        out_shape=jax.ShapeDtypeStruct((2 * EP, 1), jnp.float32),
    )(prows)


def _ln(t, g, b):
    m = jnp.mean(t, axis=-1, keepdims=True)
    v = jnp.mean((t - m) * (t - m), axis=-1, keepdims=True)
    return (t - m) * lax.rsqrt(v + 1e-5) * g + b


def _proj_body(x_ref, Win_ref, bin_ref, lA_ref, lB_ref, g_ref, b_ref, out_ref):
    x = x_ref[...]
    base = jnp.dot(x, Win_ref[...], preferred_element_type=jnp.float32)
    ada = jnp.dot(jnp.dot(x, lA_ref[...], preferred_element_type=jnp.float32),
                  lB_ref[...], preferred_element_type=jnp.float32)
    t = base + ada + bin_ref[...]
    out_ref[...] = _ln(t, g_ref[...], b_ref[...])


def _proj_tc(x, W_in, b_in, lora_A, lora_B, pn_g, pn_b):
    return pl.pallas_call(
        _proj_body,
        out_shape=jax.ShapeDtypeStruct((N, H), jnp.float32),
    )(x, W_in, b_in.reshape(1, H), lora_A, lora_B,
      pn_g.reshape(1, H), pn_b.reshape(1, H))


def _layer_body(residual, h_ref, agg_ref, deg_ref, Ws_ref, Wn_ref, bb_ref,
                g_ref, b_ref, out_ref):
    h = h_ref[...]
    agg = agg_ref[0] + agg_ref[1]
    deg = deg_ref[0] + deg_ref[1]
    agg = agg / jnp.maximum(deg, 1.0)
    t = (jnp.dot(h, Ws_ref[...], preferred_element_type=jnp.float32)
         + jnp.dot(agg, Wn_ref[...], preferred_element_type=jnp.float32)
         + bb_ref[...])
    t = jnp.maximum(t, 0.0)
    if residual:
        t = t + h
    out_ref[...] = _ln(t, g_ref[...], b_ref[...])


def _layer_tc(h, aggp, degp, Ws, Wn, bb, ln_g, ln_b, residual):
    return pl.pallas_call(
        functools.partial(_layer_body, residual),
        out_shape=jax.ShapeDtypeStruct((N, H), jnp.float32),
    )(h, aggp, degp.reshape(NC, N, 1), Ws, Wn, bb.reshape(1, H),
      ln_g.reshape(1, H), ln_b.reshape(1, H))


def kernel(x, edge_index_l0, edge_index_l1, pos_edge_index, neg_edge_index,
           W_in, b_in, lora_A, lora_B, pn_g, pn_b,
           Ws0, Wn0, bb0, ln0_g, ln0_b,
           Ws1, Wn1, bb1, ln1_g, ln1_b):
    h0 = _proj_tc(x, W_in, b_in, lora_A, lora_B, pn_g, pn_b)
    aggp0, degp0 = _sc_aggregate(h0, edge_index_l0[0], edge_index_l0[1])
    h1 = _layer_tc(h0, aggp0, degp0.reshape(NC, N, 1),
                   Ws0, Wn0, bb0, ln0_g, ln0_b, residual=False)
    aggp1, degp1 = _sc_aggregate(h1, edge_index_l1[0], edge_index_l1[1])
    h2 = _layer_tc(h1, aggp1, degp1.reshape(NC, N, 1),
                   Ws1, Wn1, bb1, ln1_g, ln1_b, residual=True)
    comb = jnp.concatenate(
        [pos_edge_index.reshape(-1), neg_edge_index.reshape(-1)])
    prows = _sc_scores(h2, comb)
    scores = _dots_tc(prows)[:, 0]
    return (scores[:EP], scores[EP:])


# speedup vs baseline: 7.0913x; 1.4826x over previous
import functools

import jax
import jax.numpy as jnp
from jax import lax
from jax.experimental import pallas as pl
from jax.experimental.pallas import tpu as pltpu
from jax.experimental.pallas import tpu_sc as plsc

N = 10000
D = 128
H = 128
E = 320000
EP = 10000

NC = 2
NS = 16
NW = NC * NS


KA = 40
EDGES_PER_TILE = E // NW
ACHUNKS = EDGES_PER_TILE // KA
NBUF = 2
NGROUPS = ACHUNKS // NBUF
NWCH = N // KA


def _sc_aggregate_body(h_hbm, src_hbm, dst_hbm, agg_out, deg_out,
                       srcA, dstA, srcB, dstB, ones_v,
                       rows0, rows1, zdeg_v,
                       agg_sh, deg_sh, sem_g, sem_s, sem_i, sem_w):
    c = lax.axis_index("c")
    s = lax.axis_index("s")
    rows = [rows0, rows1]

    gbase = (c * NS + s) * NGROUPS
    pltpu.async_copy(src_hbm.at[gbase], srcA, sem_i)
    pltpu.async_copy(dst_hbm.at[gbase], dstA, sem_i)

    zero16 = jnp.zeros((16,), jnp.float32)
    one16 = jnp.ones((16,), jnp.float32)
    for j in range(48 // 16):
        ones_v[pl.ds(j * 16, 16)] = one16
    for r in range(KA):
        for q in range(H // 16):
            rows0[r, pl.ds(q * 16, 16)] = zero16
    for j in range(2000 // 16):
        zdeg_v[pl.ds(j * 16, 16)] = zero16

    nz = (NWCH + NS - 1) // NS

    def zfire(k, carry):
        cid = s + k * NS

        @pl.when(cid < NWCH)
        def _():
            pltpu.async_copy(rows0, agg_sh.at[pl.ds(cid * KA, KA)], sem_s)

        return carry

    def zdrain(k, carry):
        cid = s + k * NS

        @pl.when(cid < NWCH)
        def _():
            pltpu.make_async_copy(
                rows0, agg_sh.at[pl.ds(cid * KA, KA)], sem_s).wait()

        return carry

    lax.fori_loop(0, nz, zfire, 0)

    @pl.when(s < 5)
    def _zero_deg():
        pltpu.async_copy(zdeg_v, deg_sh.at[pl.ds(s * 2000, 2000)], sem_s)

    lax.fori_loop(0, nz, zdrain, 0)

    @pl.when(s < 5)
    def _zero_deg_wait():
        pltpu.make_async_copy(
            zdeg_v, deg_sh.at[pl.ds(s * 2000, 2000)], sem_s).wait()

    pltpu.make_async_copy(src_hbm.at[gbase], srcA, sem_i).wait()
    pltpu.make_async_copy(dst_hbm.at[gbase], dstA, sem_i).wait()
    plsc.subcore_barrier()

    def process(g0, cur_src, cur_dst, nxt_src, nxt_dst, has_prev, has_next):
        if has_prev:
            for b in range(NBUF):
                pltpu.make_async_copy(
                    rows[b], agg_sh.at[cur_dst.at[b]], sem_s).wait()
                pltpu.make_async_copy(
                    ones_v.at[pl.ds(0, KA)], deg_sh.at[cur_dst.at[b]],
                    sem_s).wait()

        if has_next:
            pltpu.async_copy(src_hbm.at[gbase + g0 + 1], nxt_src, sem_i)
            pltpu.async_copy(dst_hbm.at[gbase + g0 + 1], nxt_dst, sem_i)

        if has_prev:
            pltpu.make_async_copy(
                src_hbm.at[gbase + g0], cur_src, sem_i).wait()
            pltpu.make_async_copy(
                dst_hbm.at[gbase + g0], cur_dst, sem_i).wait()

        for b in range(NBUF):
            pltpu.async_copy(h_hbm.at[srcs_slice(cur_src, b)], rows[b],
                             sem_g)
        for b in range(NBUF):
            pltpu.make_async_copy(
                h_hbm.at[srcs_slice(cur_src, b)], rows[b], sem_g).wait()
            pltpu.async_copy(rows[b], agg_sh.at[cur_dst.at[b]], sem_s,
                             add=True)
            pltpu.async_copy(ones_v.at[pl.ds(0, KA)],
                             deg_sh.at[cur_dst.at[b]], sem_s, add=True)

    def srcs_slice(cur_src, b):
        return cur_src.at[b]

    process(0, srcA, dstA, srcB, dstB, False, True)

    def pairB(p, carry):
        g0 = 2 * p + 1
        process(g0, srcB, dstB, srcA, dstA, True, True)
        process(g0 + 1, srcA, dstA, srcB, dstB, True, True)
        return carry

    lax.fori_loop(0, (NGROUPS - 3) // 2, pairB, 0)
    process(NGROUPS - 2, srcB, dstB, srcA, dstA, True, True)
    process(NGROUPS - 1, srcA, dstA, srcB, dstB, True, False)

    for b in range(NBUF):
        pltpu.make_async_copy(rows[b], agg_sh.at[dstA.at[b]], sem_s).wait()
        pltpu.make_async_copy(ones_v.at[pl.ds(0, KA)],
                              deg_sh.at[dstA.at[b]], sem_s).wait()

    plsc.subcore_barrier()

    nw_rounds = (NWCH + NS * NBUF - 1) // (NS * NBUF)

    def wround(r, carry):
        for b in range(NBUF):
            cid = s + (r * NBUF + b) * NS

            @pl.when(cid < NWCH)
            def _():
                pltpu.async_copy(agg_sh.at[pl.ds(cid * KA, KA)],
                                 rows[b], sem_g)

        for b in range(NBUF):
            cid = s + (r * NBUF + b) * NS

            @pl.when(cid < NWCH)
            def _():
                pltpu.make_async_copy(agg_sh.at[pl.ds(cid * KA, KA)],
                                      rows[b], sem_g).wait()
                pltpu.async_copy(rows[b],
                                 agg_out.at[c, pl.ds(cid * KA, KA)],
                                 sem_w)

        for b in range(NBUF):
            cid = s + (r * NBUF + b) * NS

            @pl.when(cid < NWCH)
            def _():
                pltpu.make_async_copy(
                    rows[b], agg_out.at[c, pl.ds(cid * KA, KA)],
                    sem_w).wait()

        return carry

    lax.fori_loop(0, nw_rounds, wround, 0)

    @pl.when(s < 5)
    def _write_deg():
        pltpu.sync_copy(deg_sh.at[pl.ds(s * 2000, 2000)], zdeg_v)
        pltpu.sync_copy(zdeg_v, deg_out.at[pl.ds(c * N + s * 2000, 2000)])


def _sc_aggregate(h, src_arr, dst_arr):
    mesh = plsc.VectorSubcoreMesh(core_axis_name="c", subcore_axis_name="s")
    src3 = src_arr.reshape(NW * NGROUPS, NBUF, KA)
    dst3 = dst_arr.reshape(NW * NGROUPS, NBUF, KA)
    return pl.kernel(
        _sc_aggregate_body,
        out_type=[jax.ShapeDtypeStruct((NC, N, H), jnp.float32),
                  jax.ShapeDtypeStruct((NC * N,), jnp.float32)],
        mesh=mesh,
        scratch_types=[
            pltpu.VMEM((NBUF, KA), jnp.int32),
            pltpu.VMEM((NBUF, KA), jnp.int32),
            pltpu.VMEM((NBUF, KA), jnp.int32),
            pltpu.VMEM((NBUF, KA), jnp.int32),
            pltpu.VMEM((48,), jnp.float32),
            pltpu.VMEM((KA, H), jnp.float32),
            pltpu.VMEM((KA, H), jnp.float32),
            pltpu.VMEM((2000,), jnp.float32),
            pltpu.VMEM_SHARED((N, H), jnp.float32),
            pltpu.VMEM_SHARED((N,), jnp.float32),
            pltpu.SemaphoreType.DMA,
            pltpu.SemaphoreType.DMA,
            pltpu.SemaphoreType.DMA,
            pltpu.SemaphoreType.DMA,
        ],
    )(h, src3, dst3)


K = 80
NCHUNKS_SCORE = 2 * EP // K
CHUNKS_PER_ARRAY = EP // K


def _sc_scores_body(h_hbm, comb_hbm, prows_out,
                    aidx_v, bidx_v, rows_a, rows_b, sem_a, sem_b):
    c = lax.axis_index("c")
    s = lax.axis_index("s")
    w = s * NC + c

    def do_chunk(cid):
        g = cid // CHUNKS_PER_ARRAY
        off = (cid % CHUNKS_PER_ARRAY) * K
        pltpu.sync_copy(comb_hbm.at[pl.ds(2 * g * EP + off, K)], aidx_v)
        pltpu.sync_copy(comb_hbm.at[pl.ds((2 * g + 1) * EP + off, K)], bidx_v)
        cp_a = pltpu.async_copy(h_hbm.at[aidx_v], rows_a, sem_a)
        cp_b = pltpu.async_copy(h_hbm.at[bidx_v], rows_b, sem_b)
        cp_a.wait()
        cp_b.wait()

        def ebody(e, carry):
            for q in range(H // 16):
                sl = pl.ds(q * 16, 16)
                rows_a[e, sl] = rows_a[e, sl] * rows_b[e, sl]
            return carry

        lax.fori_loop(0, K, ebody, 0)
        pltpu.sync_copy(rows_a, prows_out.at[pl.ds(g * EP + off, K)])

    def loop(k, carry):
        cid = w + k * NW

        @pl.when(cid < NCHUNKS_SCORE)
        def _():
            do_chunk(cid)

        return carry

    nmax = (NCHUNKS_SCORE + NW - 1) // NW
    lax.fori_loop(0, nmax, loop, 0)


def _sc_scores(h, comb):
    mesh = plsc.VectorSubcoreMesh(core_axis_name="c", subcore_axis_name="s")
    return pl.kernel(
        _sc_scores_body,
        out_type=jax.ShapeDtypeStruct((2 * EP, H), jnp.float32),
        mesh=mesh,
        scratch_types=[
            pltpu.VMEM((K,), jnp.int32),
            pltpu.VMEM((K,), jnp.int32),
            pltpu.VMEM((K, H), jnp.float32),
            pltpu.VMEM((K, H), jnp.float32),
            pltpu.SemaphoreType.DMA,
            pltpu.SemaphoreType.DMA,
        ],
    )(h, comb)


def _dots_body(p_ref, out_ref):
    out_ref[...] = jnp.sum(p_ref[...], axis=-1, keepdims=True)


def _dots_tc(prows):
    return pl.pallas_call(
        _dots_body,
        out_shape=jax.ShapeDtypeStruct((2 * EP, 1), jnp.float32),
    )(prows)


def _ln(t, g, b):
    m = jnp.mean(t, axis=-1, keepdims=True)
    v = jnp.mean((t - m) * (t - m), axis=-1, keepdims=True)
    return (t - m) * lax.rsqrt(v + 1e-5) * g + b


def _proj_body(x_ref, Win_ref, bin_ref, lA_ref, lB_ref, g_ref, b_ref, out_ref):
    x = x_ref[...]
    base = jnp.dot(x, Win_ref[...], preferred_element_type=jnp.float32)
    ada = jnp.dot(jnp.dot(x, lA_ref[...], preferred_element_type=jnp.float32),
                  lB_ref[...], preferred_element_type=jnp.float32)
    t = base + ada + bin_ref[...]
    out_ref[...] = _ln(t, g_ref[...], b_ref[...])


def _proj_tc(x, W_in, b_in, lora_A, lora_B, pn_g, pn_b):
    return pl.pallas_call(
        _proj_body,
        out_shape=jax.ShapeDtypeStruct((N, H), jnp.float32),
    )(x, W_in, b_in.reshape(1, H), lora_A, lora_B,
      pn_g.reshape(1, H), pn_b.reshape(1, H))


def _layer_body(residual, h_ref, agg_ref, deg_ref, Ws_ref, Wn_ref, bb_ref,
                g_ref, b_ref, out_ref):
    h = h_ref[...]
    agg = agg_ref[0] + agg_ref[1]
    deg = deg_ref[0] + deg_ref[1]
    agg = agg / jnp.maximum(deg, 1.0)
    t = (jnp.dot(h, Ws_ref[...], preferred_element_type=jnp.float32)
         + jnp.dot(agg, Wn_ref[...], preferred_element_type=jnp.float32)
         + bb_ref[...])
    t = jnp.maximum(t, 0.0)
    if residual:
        t = t + h
    out_ref[...] = _ln(t, g_ref[...], b_ref[...])


def _layer_tc(h, aggp, degp, Ws, Wn, bb, ln_g, ln_b, residual):
    return pl.pallas_call(
        functools.partial(_layer_body, residual),
        out_shape=jax.ShapeDtypeStruct((N, H), jnp.float32),
    )(h, aggp, degp.reshape(NC, N, 1), Ws, Wn, bb.reshape(1, H),
      ln_g.reshape(1, H), ln_b.reshape(1, H))


def kernel(x, edge_index_l0, edge_index_l1, pos_edge_index, neg_edge_index,
           W_in, b_in, lora_A, lora_B, pn_g, pn_b,
           Ws0, Wn0, bb0, ln0_g, ln0_b,
           Ws1, Wn1, bb1, ln1_g, ln1_b):
    h0 = _proj_tc(x, W_in, b_in, lora_A, lora_B, pn_g, pn_b)
    aggp0, degp0 = _sc_aggregate(h0, edge_index_l0[0], edge_index_l0[1])
    h1 = _layer_tc(h0, aggp0, degp0.reshape(NC, N, 1),
                   Ws0, Wn0, bb0, ln0_g, ln0_b, residual=False)
    aggp1, degp1 = _sc_aggregate(h1, edge_index_l1[0], edge_index_l1[1])
    h2 = _layer_tc(h1, aggp1, degp1.reshape(NC, N, 1),
                   Ws1, Wn1, bb1, ln1_g, ln1_b, residual=True)
    comb = jnp.concatenate(
        [pos_edge_index.reshape(-1), neg_edge_index.reshape(-1)])
    prows = _sc_scores(h2, comb)
    scores = _dots_tc(prows)[:, 0]
    return (scores[:EP], scores[EP:])


# speedup vs baseline: 8.0295x; 1.1323x over previous
import functools

import jax
import jax.numpy as jnp
from jax import lax
from jax.experimental import pallas as pl
from jax.experimental.pallas import tpu as pltpu
from jax.experimental.pallas import tpu_sc as plsc

N = 10000
D = 128
H = 128
E = 320000
EP = 10000

NC = 2
NS = 16
NW = NC * NS


KA = 40
EDGES_PER_TILE = E // NW
ACHUNKS = EDGES_PER_TILE // KA
NBUF = 2
NGROUPS = ACHUNKS // NBUF
NWCH = N // KA


def _sc_aggregate_body(h_hbm, src_hbm, dst_hbm, agg_out, deg_out,
                       src0, dst0, src1, dst1, src2, dst2, src3, dst3,
                       ones_v, rows0, rows1, rows2, rows3, zdeg_v,
                       agg_sh, deg_sh,
                       sem_g, sem_s, sem_w, sem_i0, sem_i1, sem_i2, sem_i3):
    c = lax.axis_index("c")
    s = lax.axis_index("s")
    rows = [rows0, rows1, rows2, rows3]
    isrc = [src0, src1, src2, src3]
    idst = [dst0, dst1, dst2, dst3]
    isem = [sem_i0, sem_i1, sem_i2, sem_i3]

    gbase = (c * NS + s) * NGROUPS
    pltpu.async_copy(src_hbm.at[gbase], isrc[0], isem[0])
    pltpu.async_copy(dst_hbm.at[gbase], idst[0], isem[0])
    pltpu.async_copy(src_hbm.at[gbase + 1], isrc[1], isem[1])
    pltpu.async_copy(dst_hbm.at[gbase + 1], idst[1], isem[1])

    zero16 = jnp.zeros((16,), jnp.float32)
    one16 = jnp.ones((16,), jnp.float32)
    for j in range(48 // 16):
        ones_v[pl.ds(j * 16, 16)] = one16
    for r in range(KA):
        for q in range(H // 16):
            rows0[r, pl.ds(q * 16, 16)] = zero16
    for j in range(2000 // 16):
        zdeg_v[pl.ds(j * 16, 16)] = zero16

    nz = (NWCH + NS - 1) // NS

    def zfire(k, carry):
        cid = s + k * NS

        @pl.when(cid < NWCH)
        def _():
            pltpu.async_copy(rows0, agg_sh.at[pl.ds(cid * KA, KA)], sem_s)

        return carry

    def zdrain(k, carry):
        cid = s + k * NS

        @pl.when(cid < NWCH)
        def _():
            pltpu.make_async_copy(
                rows0, agg_sh.at[pl.ds(cid * KA, KA)], sem_s).wait()

        return carry

    lax.fori_loop(0, nz, zfire, 0)

    @pl.when(s < 5)
    def _zero_deg():
        pltpu.async_copy(zdeg_v, deg_sh.at[pl.ds(s * 2000, 2000)], sem_s)

    lax.fori_loop(0, nz, zdrain, 0)

    @pl.when(s < 5)
    def _zero_deg_wait():
        pltpu.make_async_copy(
            zdeg_v, deg_sh.at[pl.ds(s * 2000, 2000)], sem_s).wait()

    plsc.subcore_barrier()

    def process(g, j, do_drain, do_prefetch):
        pj = (j + 2) % 4
        pr = [rows[2 * (j % 2)], rows[2 * (j % 2) + 1]]
        if do_drain:
            for b in range(NBUF):
                pltpu.make_async_copy(
                    pr[b], agg_sh.at[idst[pj].at[b]], sem_s).wait()
                pltpu.make_async_copy(
                    ones_v.at[pl.ds(0, KA)], deg_sh.at[idst[pj].at[b]],
                    sem_s).wait()
        if do_prefetch:
            pltpu.async_copy(src_hbm.at[gbase + g + 2], isrc[pj], isem[pj])
            pltpu.async_copy(dst_hbm.at[gbase + g + 2], idst[pj], isem[pj])
        pltpu.make_async_copy(src_hbm.at[gbase + g], isrc[j], isem[j]).wait()
        pltpu.make_async_copy(dst_hbm.at[gbase + g], idst[j], isem[j]).wait()
        for b in range(NBUF):
            pltpu.async_copy(h_hbm.at[isrc[j].at[b]], pr[b], sem_g)
        for b in range(NBUF):
            pltpu.make_async_copy(
                h_hbm.at[isrc[j].at[b]], pr[b], sem_g).wait()
            pltpu.async_copy(pr[b], agg_sh.at[idst[j].at[b]], sem_s,
                             add=True)
            pltpu.async_copy(ones_v.at[pl.ds(0, KA)],
                             deg_sh.at[idst[j].at[b]], sem_s, add=True)

    process(0, 0, False, True)
    process(1, 1, False, True)

    def quad(p, carry):
        g = 2 + 4 * p
        process(g, 2, True, True)
        process(g + 1, 3, True, True)
        process(g + 2, 0, True, True)
        process(g + 3, 1, True, True)
        return carry

    lax.fori_loop(0, (NGROUPS - 5) // 4, quad, 0)
    process(NGROUPS - 3, 2, True, True)
    process(NGROUPS - 2, 3, True, False)
    process(NGROUPS - 1, 0, True, False)

    for b in range(NBUF):
        pltpu.make_async_copy(
            rows[2 + b], agg_sh.at[idst[3].at[b]], sem_s).wait()
        pltpu.make_async_copy(
            ones_v.at[pl.ds(0, KA)], deg_sh.at[idst[3].at[b]], sem_s).wait()
        pltpu.make_async_copy(
            rows[b], agg_sh.at[idst[0].at[b]], sem_s).wait()
        pltpu.make_async_copy(
            ones_v.at[pl.ds(0, KA)], deg_sh.at[idst[0].at[b]], sem_s).wait()

    plsc.subcore_barrier()

    RING = 4
    nw_rounds = (NWCH + NS * RING - 1) // (NS * RING)

    def wround(r, carry):
        for b in range(RING):
            cid = s + (r * RING + b) * NS

            @pl.when(cid < NWCH)
            def _():
                pltpu.async_copy(agg_sh.at[pl.ds(cid * KA, KA)],
                                 rows[b], sem_g)

        for b in range(RING):
            cid = s + (r * RING + b) * NS

            @pl.when(cid < NWCH)
            def _():
                pltpu.make_async_copy(agg_sh.at[pl.ds(cid * KA, KA)],
                                      rows[b], sem_g).wait()
                pltpu.async_copy(rows[b],
                                 agg_out.at[c, pl.ds(cid * KA, KA)],
                                 sem_w)

        for b in range(RING):
            cid = s + (r * RING + b) * NS

            @pl.when(cid < NWCH)
            def _():
                pltpu.make_async_copy(
                    rows[b], agg_out.at[c, pl.ds(cid * KA, KA)],
                    sem_w).wait()

        return carry

    lax.fori_loop(0, nw_rounds, wround, 0)

    @pl.when(s < 5)
    def _write_deg():
        pltpu.sync_copy(deg_sh.at[pl.ds(s * 2000, 2000)], zdeg_v)
        pltpu.sync_copy(zdeg_v, deg_out.at[pl.ds(c * N + s * 2000, 2000)])


def _sc_aggregate(h, src_arr, dst_arr):
    mesh = plsc.VectorSubcoreMesh(core_axis_name="c", subcore_axis_name="s")
    src3 = src_arr.reshape(NW * NGROUPS, NBUF, KA)
    dst3 = dst_arr.reshape(NW * NGROUPS, NBUF, KA)
    return pl.kernel(
        _sc_aggregate_body,
        out_type=[jax.ShapeDtypeStruct((NC, N, H), jnp.float32),
                  jax.ShapeDtypeStruct((NC * N,), jnp.float32)],
        mesh=mesh,
        scratch_types=[
            pltpu.VMEM((NBUF, KA), jnp.int32),
            pltpu.VMEM((NBUF, KA), jnp.int32),
            pltpu.VMEM((NBUF, KA), jnp.int32),
            pltpu.VMEM((NBUF, KA), jnp.int32),
            pltpu.VMEM((NBUF, KA), jnp.int32),
            pltpu.VMEM((NBUF, KA), jnp.int32),
            pltpu.VMEM((NBUF, KA), jnp.int32),
            pltpu.VMEM((NBUF, KA), jnp.int32),
            pltpu.VMEM((48,), jnp.float32),
            pltpu.VMEM((KA, H), jnp.float32),
            pltpu.VMEM((KA, H), jnp.float32),
            pltpu.VMEM((KA, H), jnp.float32),
            pltpu.VMEM((KA, H), jnp.float32),
            pltpu.VMEM((2000,), jnp.float32),
            pltpu.VMEM_SHARED((N, H), jnp.float32),
            pltpu.VMEM_SHARED((N,), jnp.float32),
            pltpu.SemaphoreType.DMA,
            pltpu.SemaphoreType.DMA,
            pltpu.SemaphoreType.DMA,
            pltpu.SemaphoreType.DMA,
            pltpu.SemaphoreType.DMA,
            pltpu.SemaphoreType.DMA,
            pltpu.SemaphoreType.DMA,
        ],
    )(h, src3, dst3)


K = 80
NCHUNKS_SCORE = 2 * EP // K
CHUNKS_PER_ARRAY = EP // K


def _sc_scores_body(h_hbm, comb_hbm, prows_out,
                    aidx_v, bidx_v, rows_a, rows_b, sem_a, sem_b):
    c = lax.axis_index("c")
    s = lax.axis_index("s")
    w = s * NC + c

    def do_chunk(cid):
        g = cid // CHUNKS_PER_ARRAY
        off = (cid % CHUNKS_PER_ARRAY) * K
        pltpu.sync_copy(comb_hbm.at[pl.ds(2 * g * EP + off, K)], aidx_v)
        pltpu.sync_copy(comb_hbm.at[pl.ds((2 * g + 1) * EP + off, K)], bidx_v)
        cp_a = pltpu.async_copy(h_hbm.at[aidx_v], rows_a, sem_a)
        cp_b = pltpu.async_copy(h_hbm.at[bidx_v], rows_b, sem_b)
        cp_a.wait()
        cp_b.wait()

        def ebody(e, carry):
            for q in range(H // 16):
                sl = pl.ds(q * 16, 16)
                rows_a[e, sl] = rows_a[e, sl] * rows_b[e, sl]
            return carry

        lax.fori_loop(0, K, ebody, 0)
        pltpu.sync_copy(rows_a, prows_out.at[pl.ds(g * EP + off, K)])

    def loop(k, carry):
        cid = w + k * NW

        @pl.when(cid < NCHUNKS_SCORE)
        def _():
            do_chunk(cid)

        return carry

    nmax = (NCHUNKS_SCORE + NW - 1) // NW
    lax.fori_loop(0, nmax, loop, 0)


def _sc_scores(h, comb):
    mesh = plsc.VectorSubcoreMesh(core_axis_name="c", subcore_axis_name="s")
    return pl.kernel(
        _sc_scores_body,
        out_type=jax.ShapeDtypeStruct((2 * EP, H), jnp.float32),
        mesh=mesh,
        scratch_types=[
            pltpu.VMEM((K,), jnp.int32),
            pltpu.VMEM((K,), jnp.int32),
            pltpu.VMEM((K, H), jnp.float32),
            pltpu.VMEM((K, H), jnp.float32),
            pltpu.SemaphoreType.DMA,
            pltpu.SemaphoreType.DMA,
        ],
    )(h, comb)


def _dots_body(p_ref, out_ref):
    out_ref[...] = jnp.sum(p_ref[...], axis=-1, keepdims=True)


def _dots_tc(prows):
    return pl.pallas_call(
        _dots_body,
        out_shape=jax.ShapeDtypeStruct((2 * EP, 1), jnp.float32),
    )(prows)


def _ln(t, g, b):
    m = jnp.mean(t, axis=-1, keepdims=True)
    v = jnp.mean((t - m) * (t - m), axis=-1, keepdims=True)
    return (t - m) * lax.rsqrt(v + 1e-5) * g + b


def _proj_body(x_ref, Win_ref, bin_ref, lA_ref, lB_ref, g_ref, b_ref, out_ref):
    x = x_ref[...]
    base = jnp.dot(x, Win_ref[...], preferred_element_type=jnp.float32)
    ada = jnp.dot(jnp.dot(x, lA_ref[...], preferred_element_type=jnp.float32),
                  lB_ref[...], preferred_element_type=jnp.float32)
    t = base + ada + bin_ref[...]
    out_ref[...] = _ln(t, g_ref[...], b_ref[...])


def _proj_tc(x, W_in, b_in, lora_A, lora_B, pn_g, pn_b):
    return pl.pallas_call(
        _proj_body,
        out_shape=jax.ShapeDtypeStruct((N, H), jnp.float32),
    )(x, W_in, b_in.reshape(1, H), lora_A, lora_B,
      pn_g.reshape(1, H), pn_b.reshape(1, H))


def _layer_body(residual, h_ref, agg_ref, deg_ref, Ws_ref, Wn_ref, bb_ref,
                g_ref, b_ref, out_ref):
    h = h_ref[...]
    agg = agg_ref[0] + agg_ref[1]
    deg = deg_ref[0] + deg_ref[1]
    agg = agg / jnp.maximum(deg, 1.0)
    t = (jnp.dot(h, Ws_ref[...], preferred_element_type=jnp.float32)
         + jnp.dot(agg, Wn_ref[...], preferred_element_type=jnp.float32)
         + bb_ref[...])
    t = jnp.maximum(t, 0.0)
    if residual:
        t = t + h
    out_ref[...] = _ln(t, g_ref[...], b_ref[...])


def _layer_tc(h, aggp, degp, Ws, Wn, bb, ln_g, ln_b, residual):
    return pl.pallas_call(
        functools.partial(_layer_body, residual),
        out_shape=jax.ShapeDtypeStruct((N, H), jnp.float32),
    )(h, aggp, degp.reshape(NC, N, 1), Ws, Wn, bb.reshape(1, H),
      ln_g.reshape(1, H), ln_b.reshape(1, H))


def kernel(x, edge_index_l0, edge_index_l1, pos_edge_index, neg_edge_index,
           W_in, b_in, lora_A, lora_B, pn_g, pn_b,
           Ws0, Wn0, bb0, ln0_g, ln0_b,
           Ws1, Wn1, bb1, ln1_g, ln1_b):
    h0 = _proj_tc(x, W_in, b_in, lora_A, lora_B, pn_g, pn_b)
    aggp0, degp0 = _sc_aggregate(h0, edge_index_l0[0], edge_index_l0[1])
    h1 = _layer_tc(h0, aggp0, degp0.reshape(NC, N, 1),
                   Ws0, Wn0, bb0, ln0_g, ln0_b, residual=False)
    aggp1, degp1 = _sc_aggregate(h1, edge_index_l1[0], edge_index_l1[1])
    h2 = _layer_tc(h1, aggp1, degp1.reshape(NC, N, 1),
                   Ws1, Wn1, bb1, ln1_g, ln1_b, residual=True)
    comb = jnp.concatenate(
        [pos_edge_index.reshape(-1), neg_edge_index.reshape(-1)])
    prows = _sc_scores(h2, comb)
    scores = _dots_tc(prows)[:, 0]
    return (scores[:EP], scores[EP:])


# speedup vs baseline: 8.3179x; 1.0359x over previous
import functools

import jax
import jax.numpy as jnp
from jax import lax
from jax.experimental import pallas as pl
from jax.experimental.pallas import tpu as pltpu
from jax.experimental.pallas import tpu_sc as plsc

N = 10000
D = 128
H = 128
E = 320000
EP = 10000

NC = 2
NS = 16
NW = NC * NS


KA = 80
EDGES_PER_TILE = E // NW
NGROUPS = EDGES_PER_TILE // KA
NWCH = N // KA
NRING = 3


def _sc_aggregate_body(h_hbm, src_hbm, dst_hbm, agg_out, deg_out,
                       src0, dst0, src1, dst1, src2, dst2,
                       ones_v, rows0, rows1, rows2, zdeg_v,
                       agg_sh, deg_sh,
                       sem_g, sem_s, sem_w, sem_i0, sem_i1, sem_i2):
    c = lax.axis_index("c")
    s = lax.axis_index("s")
    rows = [rows0, rows1, rows2]
    isrc = [src0, src1, src2]
    idst = [dst0, dst1, dst2]
    isem = [sem_i0, sem_i1, sem_i2]

    gbase = (c * NS + s) * NGROUPS
    pltpu.async_copy(src_hbm.at[gbase], isrc[0], isem[0])
    pltpu.async_copy(dst_hbm.at[gbase], idst[0], isem[0])

    zero16 = jnp.zeros((16,), jnp.float32)
    one16 = jnp.ones((16,), jnp.float32)
    for j in range(KA // 16):
        ones_v[pl.ds(j * 16, 16)] = one16
    for r in range(KA):
        for q in range(H // 16):
            rows0[r, pl.ds(q * 16, 16)] = zero16
    for j in range(2000 // 16):
        zdeg_v[pl.ds(j * 16, 16)] = zero16

    nz = (NWCH + NS - 1) // NS

    def zfire(k, carry):
        cid = s + k * NS

        @pl.when(cid < NWCH)
        def _():
            pltpu.async_copy(rows0, agg_sh.at[pl.ds(cid * KA, KA)], sem_s)

        return carry

    def zdrain(k, carry):
        cid = s + k * NS

        @pl.when(cid < NWCH)
        def _():
            pltpu.make_async_copy(
                rows0, agg_sh.at[pl.ds(cid * KA, KA)], sem_s).wait()

        return carry

    lax.fori_loop(0, nz, zfire, 0)

    @pl.when(s < 5)
    def _zero_deg():
        pltpu.async_copy(zdeg_v, deg_sh.at[pl.ds(s * 2000, 2000)], sem_s)

    lax.fori_loop(0, nz, zdrain, 0)

    @pl.when(s < 5)
    def _zero_deg_wait():
        pltpu.make_async_copy(
            zdeg_v, deg_sh.at[pl.ds(s * 2000, 2000)], sem_s).wait()

    plsc.subcore_barrier()

    def process(g, j, do_drain, do_prefetch):
        pj = (j + 1) % NRING
        if do_drain:
            pltpu.make_async_copy(
                rows[pj], agg_sh.at[idst[pj].at[0]], sem_s).wait()
            pltpu.make_async_copy(
                ones_v, deg_sh.at[idst[pj].at[0]], sem_s).wait()
        if do_prefetch:
            pltpu.async_copy(src_hbm.at[gbase + g + 1], isrc[pj], isem[pj])
            pltpu.async_copy(dst_hbm.at[gbase + g + 1], idst[pj], isem[pj])
        pltpu.make_async_copy(src_hbm.at[gbase + g], isrc[j], isem[j]).wait()
        pltpu.make_async_copy(dst_hbm.at[gbase + g], idst[j], isem[j]).wait()
        pltpu.async_copy(h_hbm.at[isrc[j].at[0]], rows[j], sem_g)
        pltpu.make_async_copy(h_hbm.at[isrc[j].at[0]], rows[j], sem_g).wait()
        pltpu.async_copy(rows[j], agg_sh.at[idst[j].at[0]], sem_s, add=True)
        pltpu.async_copy(ones_v, deg_sh.at[idst[j].at[0]], sem_s, add=True)

    process(0, 0, False, True)
    process(1, 1, False, True)
    process(2, 2, True, True)

    def triple(p, carry):
        g = 3 + 3 * p
        process(g, 0, True, True)
        process(g + 1, 1, True, True)
        process(g + 2, 2, True, True)
        return carry

    lax.fori_loop(0, (NGROUPS - 5) // 3, triple, 0)
    process(NGROUPS - 2, 0, True, True)
    process(NGROUPS - 1, 1, True, False)

    for j in (0, 1):
        pltpu.make_async_copy(
            rows[j], agg_sh.at[idst[j].at[0]], sem_s).wait()
        pltpu.make_async_copy(
            ones_v, deg_sh.at[idst[j].at[0]], sem_s).wait()

    plsc.subcore_barrier()

    nw_rounds = (NWCH + NS * NRING - 1) // (NS * NRING)

    def wround(r, carry):
        for b in range(NRING):
            cid = s + (r * NRING + b) * NS

            @pl.when(cid < NWCH)
            def _():
                pltpu.async_copy(agg_sh.at[pl.ds(cid * KA, KA)],
                                 rows[b], sem_g)

        for b in range(NRING):
            cid = s + (r * NRING + b) * NS

            @pl.when(cid < NWCH)
            def _():
                pltpu.make_async_copy(agg_sh.at[pl.ds(cid * KA, KA)],
                                      rows[b], sem_g).wait()
                pltpu.async_copy(rows[b],
                                 agg_out.at[c, pl.ds(cid * KA, KA)],
                                 sem_w)

        for b in range(NRING):
            cid = s + (r * NRING + b) * NS

            @pl.when(cid < NWCH)
            def _():
                pltpu.make_async_copy(
                    rows[b], agg_out.at[c, pl.ds(cid * KA, KA)],
                    sem_w).wait()

        return carry

    lax.fori_loop(0, nw_rounds, wround, 0)

    @pl.when(s < 5)
    def _write_deg():
        pltpu.sync_copy(deg_sh.at[pl.ds(s * 2000, 2000)], zdeg_v)
        pltpu.sync_copy(zdeg_v, deg_out.at[pl.ds(c * N + s * 2000, 2000)])


def _sc_aggregate(h, src_arr, dst_arr):
    mesh = plsc.VectorSubcoreMesh(core_axis_name="c", subcore_axis_name="s")
    src3 = src_arr.reshape(NW * NGROUPS, 1, KA)
    dst3 = dst_arr.reshape(NW * NGROUPS, 1, KA)
    return pl.kernel(
        _sc_aggregate_body,
        out_type=[jax.ShapeDtypeStruct((NC, N, H), jnp.float32),
                  jax.ShapeDtypeStruct((NC * N,), jnp.float32)],
        mesh=mesh,
        scratch_types=[
            pltpu.VMEM((1, KA), jnp.int32),
            pltpu.VMEM((1, KA), jnp.int32),
            pltpu.VMEM((1, KA), jnp.int32),
            pltpu.VMEM((1, KA), jnp.int32),
            pltpu.VMEM((1, KA), jnp.int32),
            pltpu.VMEM((1, KA), jnp.int32),
            pltpu.VMEM((KA,), jnp.float32),
            pltpu.VMEM((KA, H), jnp.float32),
            pltpu.VMEM((KA, H), jnp.float32),
            pltpu.VMEM((KA, H), jnp.float32),
            pltpu.VMEM((2000,), jnp.float32),
            pltpu.VMEM_SHARED((N, H), jnp.float32),
            pltpu.VMEM_SHARED((N,), jnp.float32),
            pltpu.SemaphoreType.DMA,
            pltpu.SemaphoreType.DMA,
            pltpu.SemaphoreType.DMA,
            pltpu.SemaphoreType.DMA,
            pltpu.SemaphoreType.DMA,
            pltpu.SemaphoreType.DMA,
        ],
    )(h, src3, dst3)


K = 80
NCHUNKS_SCORE = 2 * EP // K
CHUNKS_PER_ARRAY = EP // K


def _sc_scores_body(h_hbm, comb_hbm, prows_out,
                    aidx_v, bidx_v, rows_a, rows_b, sem_a, sem_b):
    c = lax.axis_index("c")
    s = lax.axis_index("s")
    w = s * NC + c

    def do_chunk(cid):
        g = cid // CHUNKS_PER_ARRAY
        off = (cid % CHUNKS_PER_ARRAY) * K
        pltpu.sync_copy(comb_hbm.at[pl.ds(2 * g * EP + off, K)], aidx_v)
        pltpu.sync_copy(comb_hbm.at[pl.ds((2 * g + 1) * EP + off, K)], bidx_v)
        cp_a = pltpu.async_copy(h_hbm.at[aidx_v], rows_a, sem_a)
        cp_b = pltpu.async_copy(h_hbm.at[bidx_v], rows_b, sem_b)
        cp_a.wait()
        cp_b.wait()

        def ebody(e, carry):
            for q in range(H // 16):
                sl = pl.ds(q * 16, 16)
                rows_a[e, sl] = rows_a[e, sl] * rows_b[e, sl]
            return carry

        lax.fori_loop(0, K, ebody, 0)
        pltpu.sync_copy(rows_a, prows_out.at[pl.ds(g * EP + off, K)])

    def loop(k, carry):
        cid = w + k * NW

        @pl.when(cid < NCHUNKS_SCORE)
        def _():
            do_chunk(cid)

        return carry

    nmax = (NCHUNKS_SCORE + NW - 1) // NW
    lax.fori_loop(0, nmax, loop, 0)


def _sc_scores(h, comb):
    mesh = plsc.VectorSubcoreMesh(core_axis_name="c", subcore_axis_name="s")
    return pl.kernel(
        _sc_scores_body,
        out_type=jax.ShapeDtypeStruct((2 * EP, H), jnp.float32),
        mesh=mesh,
        scratch_types=[
            pltpu.VMEM((K,), jnp.int32),
            pltpu.VMEM((K,), jnp.int32),
            pltpu.VMEM((K, H), jnp.float32),
            pltpu.VMEM((K, H), jnp.float32),
            pltpu.SemaphoreType.DMA,
            pltpu.SemaphoreType.DMA,
        ],
    )(h, comb)


def _dots_body(p_ref, out_ref):
    out_ref[...] = jnp.sum(p_ref[...], axis=-1, keepdims=True)


def _dots_tc(prows):
    return pl.pallas_call(
        _dots_body,
        out_shape=jax.ShapeDtypeStruct((2 * EP, 1), jnp.float32),
    )(prows)


def _ln(t, g, b):
    m = jnp.mean(t, axis=-1, keepdims=True)
    v = jnp.mean((t - m) * (t - m), axis=-1, keepdims=True)
    return (t - m) * lax.rsqrt(v + 1e-5) * g + b


def _proj_body(x_ref, Win_ref, bin_ref, lA_ref, lB_ref, g_ref, b_ref, out_ref):
    x = x_ref[...]
    base = jnp.dot(x, Win_ref[...], preferred_element_type=jnp.float32)
    ada = jnp.dot(jnp.dot(x, lA_ref[...], preferred_element_type=jnp.float32),
                  lB_ref[...], preferred_element_type=jnp.float32)
    t = base + ada + bin_ref[...]
    out_ref[...] = _ln(t, g_ref[...], b_ref[...])


def _proj_tc(x, W_in, b_in, lora_A, lora_B, pn_g, pn_b):
    return pl.pallas_call(
        _proj_body,
        out_shape=jax.ShapeDtypeStruct((N, H), jnp.float32),
    )(x, W_in, b_in.reshape(1, H), lora_A, lora_B,
      pn_g.reshape(1, H), pn_b.reshape(1, H))


def _layer_body(residual, h_ref, agg_ref, deg_ref, Ws_ref, Wn_ref, bb_ref,
                g_ref, b_ref, out_ref):
    h = h_ref[...]
    agg = agg_ref[0] + agg_ref[1]
    deg = deg_ref[0] + deg_ref[1]
    agg = agg / jnp.maximum(deg, 1.0)
    t = (jnp.dot(h, Ws_ref[...], preferred_element_type=jnp.float32)
         + jnp.dot(agg, Wn_ref[...], preferred_element_type=jnp.float32)
         + bb_ref[...])
    t = jnp.maximum(t, 0.0)
    if residual:
        t = t + h
    out_ref[...] = _ln(t, g_ref[...], b_ref[...])


def _layer_tc(h, aggp, degp, Ws, Wn, bb, ln_g, ln_b, residual):
    return pl.pallas_call(
        functools.partial(_layer_body, residual),
        out_shape=jax.ShapeDtypeStruct((N, H), jnp.float32),
    )(h, aggp, degp.reshape(NC, N, 1), Ws, Wn, bb.reshape(1, H),
      ln_g.reshape(1, H), ln_b.reshape(1, H))


def kernel(x, edge_index_l0, edge_index_l1, pos_edge_index, neg_edge_index,
           W_in, b_in, lora_A, lora_B, pn_g, pn_b,
           Ws0, Wn0, bb0, ln0_g, ln0_b,
           Ws1, Wn1, bb1, ln1_g, ln1_b):
    h0 = _proj_tc(x, W_in, b_in, lora_A, lora_B, pn_g, pn_b)
    aggp0, degp0 = _sc_aggregate(h0, edge_index_l0[0], edge_index_l0[1])
    h1 = _layer_tc(h0, aggp0, degp0.reshape(NC, N, 1),
                   Ws0, Wn0, bb0, ln0_g, ln0_b, residual=False)
    aggp1, degp1 = _sc_aggregate(h1, edge_index_l1[0], edge_index_l1[1])
    h2 = _layer_tc(h1, aggp1, degp1.reshape(NC, N, 1),
                   Ws1, Wn1, bb1, ln1_g, ln1_b, residual=True)
    comb = jnp.concatenate(
        [pos_edge_index.reshape(-1), neg_edge_index.reshape(-1)])
    prows = _sc_scores(h2, comb)
    scores = _dots_tc(prows)[:, 0]
    return (scores[:EP], scores[EP:])
